# baseline (speedup 1.0000x reference)
import jax
import jax.numpy as jnp
from jax.experimental import pallas as pl
from jax.experimental.pallas import tpu as pltpu

B, T, C = 16, 2048, 512
G, V, VD = 2, 320, 256
N = B * T
BLK = 2048


def _body(x_ref, gu_ref, W_ref, b_ref, cv_ref, out_ref):
    x = x_ref[...]
    logits = jnp.dot(x, W_ref[...], preferred_element_type=jnp.float32)
    logits = logits + b_ref[...]
    eps = 1e-10
    u = gu_ref[...] * (1.0 - 2.0 * eps) + eps
    y = logits - jnp.log(-jnp.log(u))
    for g in range(G):
        yg = y[:, g * V:(g + 1) * V]
        m = jnp.max(yg, axis=1, keepdims=True)
        iota = jax.lax.broadcasted_iota(jnp.int32, (BLK, V), 1)
        first = jnp.min(jnp.where(yg == m, iota, V), axis=1, keepdims=True)
        oh = (iota == first).astype(jnp.float32)
        qg = jnp.dot(oh, cv_ref[g], preferred_element_type=jnp.float32)
        out_ref[:, g * VD:(g + 1) * VD] = qg


def kernel(x, gumbel_u, W, b, codevectors):
    x2 = x.reshape(N, C)
    gu2 = gumbel_u.reshape(N, G * V)
    cv3 = codevectors.reshape(G, V, VD)
    grid = (N // BLK,)
    out = pl.pallas_call(
        _body,
        grid=grid,
        in_specs=[
            pl.BlockSpec((BLK, C), lambda i: (i, 0)),
            pl.BlockSpec((BLK, G * V), lambda i: (i, 0)),
            pl.BlockSpec((C, G * V), lambda i: (0, 0)),
            pl.BlockSpec((1, G * V), lambda i: (0, 0)),
            pl.BlockSpec((G, V, VD), lambda i: (0, 0, 0)),
        ],
        out_specs=pl.BlockSpec((BLK, G * VD), lambda i: (i, 0)),
        out_shape=jax.ShapeDtypeStruct((N, G * VD), jnp.float32),
    )(x2, gu2, W, b.reshape(1, G * V), cv3)
    return out.reshape(B, T, G * VD)


# speedup vs baseline: 1.0155x; 1.0155x over previous
import jax
import jax.numpy as jnp
from jax.experimental import pallas as pl
from jax.experimental.pallas import tpu as pltpu

B, T, C = 16, 2048, 512
G, V, VD = 2, 320, 256
N = B * T
BLK = 2048


def _body(x_ref, gu_ref, W_ref, b_ref, cv_ref, out_ref):
    x = x_ref[...]
    logits = jnp.dot(x, W_ref[...], preferred_element_type=jnp.float32)
    logits = logits + b_ref[...]
    eps = 1e-10
    u = gu_ref[...] * (1.0 - 2.0 * eps) + eps
    y = logits - jnp.log(-jnp.log(u))
    for g in range(G):
        yg = y[:, g * V:(g + 1) * V]
        m = jnp.max(yg, axis=1, keepdims=True)
        iota = jax.lax.broadcasted_iota(jnp.int32, (BLK, V), 1)
        first = jnp.min(jnp.where(yg == m, iota, V), axis=1, keepdims=True)
        oh = (iota == first).astype(jnp.float32)
        qg = jnp.dot(oh, cv_ref[g * V:(g + 1) * V, :],
                     preferred_element_type=jnp.float32)
        out_ref[:, g * VD:(g + 1) * VD] = qg


def kernel(x, gumbel_u, W, b, codevectors):
    x2 = x.reshape(N, C)
    gu2 = gumbel_u.reshape(N, G * V)
    grid = (N // BLK,)
    out = pl.pallas_call(
        _body,
        grid=grid,
        in_specs=[
            pl.BlockSpec((BLK, C), lambda i: (i, 0)),
            pl.BlockSpec((BLK, G * V), lambda i: (i, 0)),
            pl.BlockSpec((C, G * V), lambda i: (0, 0)),
            pl.BlockSpec((G * V,), lambda i: (0,)),
            pl.BlockSpec((G * V, VD), lambda i: (0, 0)),
        ],
        out_specs=pl.BlockSpec((BLK, G * VD), lambda i: (i, 0)),
        out_shape=jax.ShapeDtypeStruct((N, G * VD), jnp.float32),
    )(x2, gu2, W, b, codevectors)
    return out.reshape(B, T, G * VD)
